# initial kernel scaffold (unmeasured)
import jax
import jax.numpy as jnp
from jax import lax
from jax.experimental import pallas as pl
from jax.experimental.pallas import tpu as pltpu


def kernel(
    x,
):
    def body(*refs):
        pass

    out_shape = jax.ShapeDtypeStruct(..., jnp.float32)
    return pl.pallas_call(body, out_shape=out_shape)(...)


# baseline (device time: 1098234 ns/iter reference)
import jax
import jax.numpy as jnp
from jax import lax
from jax.experimental import pallas as pl
from jax.experimental.pallas import tpu as pltpu

X_SIZE = 2


def kernel(x):
    m, n = x.shape
    xb = x.astype(jnp.bfloat16)

    def body(x_ref, out_ref, copy_sem, send_sem, recv_sem):
        my_x = lax.axis_index("x")
        my_y = lax.axis_index("y")
        nbr = (1 - my_x, my_y)

        barrier_sem = pltpu.get_barrier_semaphore()
        pl.semaphore_signal(
            barrier_sem, inc=1, device_id=nbr,
            device_id_type=pl.DeviceIdType.MESH,
        )
        pl.semaphore_wait(barrier_sem, 1)

        local = pltpu.make_async_copy(
            x_ref, out_ref.at[pl.ds(my_x * m, m), :], copy_sem
        )
        local.start()

        rdma = pltpu.make_async_remote_copy(
            src_ref=x_ref,
            dst_ref=out_ref.at[pl.ds(my_x * m, m), :],
            send_sem=send_sem,
            recv_sem=recv_sem,
            device_id=nbr,
            device_id_type=pl.DeviceIdType.MESH,
        )
        rdma.start()
        rdma.wait()
        local.wait()

    return pl.pallas_call(
        body,
        out_shape=jax.ShapeDtypeStruct((X_SIZE * m, n), jnp.bfloat16),
        in_specs=[pl.BlockSpec(memory_space=pltpu.MemorySpace.HBM)],
        out_specs=pl.BlockSpec(memory_space=pltpu.MemorySpace.HBM),
        scratch_shapes=[
            pltpu.SemaphoreType.DMA,
            pltpu.SemaphoreType.DMA,
            pltpu.SemaphoreType.DMA,
        ],
        compiler_params=pltpu.CompilerParams(collective_id=0),
    )(xb)


# device time: 259804 ns/iter; 4.2272x vs baseline; 4.2272x over previous
import jax
import jax.numpy as jnp
from jax import lax
from jax.experimental import pallas as pl
from jax.experimental.pallas import tpu as pltpu

X_SIZE = 2
C = 1024


def kernel(x):
    m, n = x.shape
    h = m // 2
    nc = h // C

    def body(
        x_ref, out_ref,
        send_buf, recv_x_buf, recv_y_buf, f32_buf, oth_buf,
        load_sems, send_x_sems, recv_x_sems, send_y_sems, recv_y_sems,
        owncp_sem, othcp_sems, rxcp_sem, rycp_sem,
    ):
        my_x = lax.axis_index("x")
        my_y = lax.axis_index("y")
        x_nbr = (1 - my_x, my_y)
        y_nbr = (my_x, 1 - my_y)

        barrier_sem = pltpu.get_barrier_semaphore()
        for nbr in (x_nbr, y_nbr):
            pl.semaphore_signal(
                barrier_sem, inc=1, device_id=nbr,
                device_id_type=pl.DeviceIdType.MESH,
            )
        pl.semaphore_wait(barrier_sem, 2)

        own_base = my_x * m
        rem_base = (1 - my_x) * m
        send_half = my_y * h
        oth_half = (1 - my_y) * h

        def chunk_row(k):
            if k < nc:
                return send_half + k * C
            return oth_half + (k - nc) * C

        loads = []

        def start_load(k):
            d = pltpu.make_async_copy(
                x_ref.at[pl.ds(chunk_row(k), C), :],
                f32_buf.at[k % 2],
                load_sems.at[k % 2],
            )
            d.start()
            loads.append(d)

        start_load(0)
        send_rdmas = []
        owncp = []
        othcp = [None, None]
        for k in range(2 * nc):
            if k + 1 < 2 * nc:
                start_load(k + 1)
            loads[k].wait()
            if k < nc:
                send_buf[k] = f32_buf[k % 2].astype(jnp.bfloat16)
                rd = pltpu.make_async_remote_copy(
                    src_ref=send_buf.at[k],
                    dst_ref=recv_x_buf.at[k],
                    send_sem=send_x_sems.at[k],
                    recv_sem=recv_x_sems.at[k],
                    device_id=x_nbr,
                    device_id_type=pl.DeviceIdType.MESH,
                )
                rd.start()
                send_rdmas.append(rd)
                cp = pltpu.make_async_copy(
                    send_buf.at[k],
                    out_ref.at[pl.ds(own_base + send_half + k * C, C), :],
                    owncp_sem,
                )
                cp.start()
                owncp.append(cp)
            else:
                j = k - nc
                s = j % 2
                if othcp[s] is not None:
                    othcp[s].wait()
                oth_buf[s] = f32_buf[k % 2].astype(jnp.bfloat16)
                cp = pltpu.make_async_copy(
                    oth_buf.at[s],
                    out_ref.at[pl.ds(own_base + oth_half + j * C, C), :],
                    othcp_sems.at[s],
                )
                cp.start()
                othcp[s] = cp

        fwd_rdmas = []
        rxcp = []
        for f in range(nc):
            rv = pltpu.make_async_remote_copy(
                src_ref=recv_x_buf.at[f],
                dst_ref=recv_x_buf.at[f],
                send_sem=send_x_sems.at[f],
                recv_sem=recv_x_sems.at[f],
                device_id=x_nbr,
                device_id_type=pl.DeviceIdType.MESH,
            )
            rv.wait_recv()
            fw = pltpu.make_async_remote_copy(
                src_ref=recv_x_buf.at[f],
                dst_ref=recv_y_buf.at[f],
                send_sem=send_y_sems.at[f],
                recv_sem=recv_y_sems.at[f],
                device_id=y_nbr,
                device_id_type=pl.DeviceIdType.MESH,
            )
            fw.start()
            fwd_rdmas.append(fw)
            cp = pltpu.make_async_copy(
                recv_x_buf.at[f],
                out_ref.at[pl.ds(rem_base + send_half + f * C, C), :],
                rxcp_sem,
            )
            cp.start()
            rxcp.append(cp)

        rycp = []
        for f in range(nc):
            rv = pltpu.make_async_remote_copy(
                src_ref=recv_y_buf.at[f],
                dst_ref=recv_y_buf.at[f],
                send_sem=send_y_sems.at[f],
                recv_sem=recv_y_sems.at[f],
                device_id=y_nbr,
                device_id_type=pl.DeviceIdType.MESH,
            )
            rv.wait_recv()
            cp = pltpu.make_async_copy(
                recv_y_buf.at[f],
                out_ref.at[pl.ds(rem_base + oth_half + f * C, C), :],
                rycp_sem,
            )
            cp.start()
            rycp.append(cp)

        for rd in send_rdmas:
            rd.wait_send()
        for fw in fwd_rdmas:
            fw.wait_send()
        for cp in owncp:
            cp.wait()
        for cp in othcp:
            if cp is not None:
                cp.wait()
        for cp in rxcp:
            cp.wait()
        for cp in rycp:
            cp.wait()

    return pl.pallas_call(
        body,
        out_shape=jax.ShapeDtypeStruct((X_SIZE * m, n), jnp.bfloat16),
        in_specs=[pl.BlockSpec(memory_space=pltpu.MemorySpace.HBM)],
        out_specs=pl.BlockSpec(memory_space=pltpu.MemorySpace.HBM),
        scratch_shapes=[
            pltpu.VMEM((nc, C, n), jnp.bfloat16),
            pltpu.VMEM((nc, C, n), jnp.bfloat16),
            pltpu.VMEM((nc, C, n), jnp.bfloat16),
            pltpu.VMEM((2, C, n), jnp.float32),
            pltpu.VMEM((2, C, n), jnp.bfloat16),
            pltpu.SemaphoreType.DMA((2,)),
            pltpu.SemaphoreType.DMA((nc,)),
            pltpu.SemaphoreType.DMA((nc,)),
            pltpu.SemaphoreType.DMA((nc,)),
            pltpu.SemaphoreType.DMA((nc,)),
            pltpu.SemaphoreType.DMA,
            pltpu.SemaphoreType.DMA((2,)),
            pltpu.SemaphoreType.DMA,
            pltpu.SemaphoreType.DMA,
        ],
        compiler_params=pltpu.CompilerParams(
            collective_id=0,
            vmem_limit_bytes=100 * 1024 * 1024,
        ),
    )(x)
